# Initial kernel scaffold; baseline (speedup 1.0000x reference)
import jax
import jax.numpy as jnp
from jax.experimental import pallas as pl


def kernel(x, edge_index, W1, b1, W2, b2, gamma, beta):
    raise NotImplementedError("write your pallas kernel here")


# speedup vs baseline: 4.5565x; 4.5565x over previous
import functools

import jax
import jax.numpy as jnp
from jax import lax
from jax.experimental import pallas as pl
from jax.experimental.pallas import tpu as pltpu
from jax.experimental.pallas import tpu_sc as plsc

_NC = 2
_NS = 16


@functools.lru_cache(maxsize=None)
def _make_seg_sum(N, E, D):
    NW = _NC * _NS
    e_per_w = E // NW
    K = 40
    niter = e_per_w // K
    assert e_per_w % K == 0 and niter % 2 == 0
    stride = ((N // _NS) // 8) * 8
    wsize = N - (_NS - 1) * stride
    assert wsize % 8 == 0 and wsize >= stride and E % NW == 0

    mesh = plsc.VectorSubcoreMesh(core_axis_name="c", subcore_axis_name="s")

    @functools.partial(
        pl.kernel,
        out_type=jax.ShapeDtypeStruct((_NC * N, D), jnp.float32),
        mesh=mesh,
        scratch_types=[
            pltpu.VMEM((K,), jnp.int32),
            pltpu.VMEM((K,), jnp.int32),
            pltpu.VMEM((K,), jnp.int32),
            pltpu.VMEM((K,), jnp.int32),
            pltpu.VMEM((K, D), jnp.float32),
            pltpu.VMEM((K, D), jnp.float32),
            pltpu.VMEM_SHARED((N, D), jnp.float32),
            pltpu.SemaphoreType.DMA,
            pltpu.SemaphoreType.DMA,
        ],
    )
    def seg(src_hbm, dst_hbm, h_hbm, zero_hbm, out_hbm,
            src0, src1, dst0, dst1, rows0, rows1, acc, sem0, sem1):
        srcv = (src0, src1)
        dstv = (dst0, dst1)
        rows = (rows0, rows1)
        sems = (sem0, sem1)
        cid = lax.axis_index("c")
        sid = lax.axis_index("s")
        wid = cid * _NS + sid
        e_base = wid * e_per_w

        rs = pl.ds(sid * stride, wsize)

        @pl.when(cid == 0)
        def _():
            pltpu.sync_copy(h_hbm.at[rs], acc.at[rs])

        @pl.when(cid > 0)
        def _():
            pltpu.sync_copy(zero_hbm.at[rs], acc.at[rs])

        plsc.subcore_barrier()

        def load_idx(i, b):
            off = e_base + i * K
            pltpu.sync_copy(src_hbm.at[pl.ds(off, K)], srcv[b])
            pltpu.sync_copy(dst_hbm.at[pl.ds(off, K)], dstv[b])

        def fire(b):
            pltpu.async_copy(h_hbm.at[srcv[b]], rows[b], sems[b])

        load_idx(0, 0)
        fire(0)
        load_idx(1, 1)
        fire(1)

        @pl.loop(0, niter, step=2)
        def _(i):
            for b in range(2):
                ii = i + b
                pltpu.make_async_copy(h_hbm.at[srcv[b]], rows[b], sems[b]).wait()
                pltpu.sync_copy(rows[b], acc.at[dstv[b]], add=True)

                @pl.when(ii + 2 < niter)
                def _():
                    load_idx(ii + 2, b)
                    fire(b)

        plsc.subcore_barrier()
        pltpu.sync_copy(acc.at[rs], out_hbm.at[pl.ds(cid * N + sid * stride, wsize)])

    return seg


@functools.lru_cache(maxsize=None)
def _make_mlp(N, D):
    def body(agg_ref, w1_ref, b1_ref, w2_ref, b2_ref, g_ref, be_ref,
             out_ref, mean_ref):
        z = agg_ref[:N, :] + agg_ref[N:, :]
        y = lax.dot_general(z, w1_ref[...], (((1,), (1,)), ((), ())),
                            precision=lax.Precision.HIGHEST,
                            preferred_element_type=jnp.float32) + b1_ref[...]
        mu = jnp.mean(y, axis=0, keepdims=True)
        var = jnp.mean((y - mu) * (y - mu), axis=0, keepdims=True)
        r = (y - mu) * lax.rsqrt(var + 1e-5) * g_ref[...] + be_ref[...]
        r = jnp.maximum(r, 0.0)
        o = lax.dot_general(r, w2_ref[...], (((1,), (1,)), ((), ())),
                            precision=lax.Precision.HIGHEST,
                            preferred_element_type=jnp.float32) + b2_ref[...]
        out_ref[...] = o
        mean_ref[...] = jnp.mean(o, axis=0, keepdims=True)

    return pl.pallas_call(
        body,
        out_shape=(jax.ShapeDtypeStruct((N, D), jnp.float32),
                   jax.ShapeDtypeStruct((1, D), jnp.float32)),
    )


def kernel(x, edge_index, W1, b1, W2, b2, gamma, beta):
    N, D = x.shape
    E = edge_index.shape[1]
    L = W1.shape[0]
    src = edge_index[0]
    dst = edge_index[1]
    zeros = jnp.zeros((N, D), jnp.float32)
    seg = _make_seg_sum(N, E, D)
    mlp = _make_mlp(N, D)

    h = x
    means = []
    for l in range(L):
        agg2 = seg(src, dst, h, zeros)
        h, m = mlp(agg2, W1[l], b1[l].reshape(1, D), W2[l], b2[l].reshape(1, D),
                   gamma[l].reshape(1, D), beta[l].reshape(1, D))
        means.append(m.reshape(D))
    return (means[-1], means[-2])


# speedup vs baseline: 9.7148x; 2.1321x over previous
import functools

import jax
import jax.numpy as jnp
from jax import lax
from jax.experimental import pallas as pl
from jax.experimental.pallas import tpu as pltpu
from jax.experimental.pallas import tpu_sc as plsc

_NC = 2
_NS = 16


@functools.lru_cache(maxsize=None)
def _make_seg_sum(N, E, D):
    NW = _NC * _NS
    e_per_w = E // NW
    K = 80
    niter = e_per_w // K
    assert e_per_w % K == 0
    stride = ((N // _NS) // 8) * 8
    wsize = N - (_NS - 1) * stride
    assert wsize % 8 == 0 and wsize >= stride and E % NW == 0

    mesh = plsc.VectorSubcoreMesh(core_axis_name="c", subcore_axis_name="s")

    nbuf = 2
    scratch_types = [pltpu.VMEM((e_per_w,), jnp.int32),
                     pltpu.VMEM_SHARED((N, D), jnp.float32)]
    scratch_types += [pltpu.VMEM((K,), jnp.int32) for _ in range(nbuf)]
    scratch_types += [pltpu.VMEM((K, D), jnp.float32) for _ in range(nbuf)]
    scratch_types += [pltpu.SemaphoreType.DMA for _ in range(2 * nbuf)]

    @functools.partial(
        pl.kernel,
        out_type=jax.ShapeDtypeStruct((_NC * N, D), jnp.float32),
        mesh=mesh,
        scratch_types=scratch_types,
    )
    def seg(src_hbm, dst_hbm, h_hbm, zero_hbm, out_hbm, src_all, acc, *rest):
        dstv = rest[:nbuf]
        rows = rest[nbuf:2 * nbuf]
        gsem = rest[2 * nbuf:3 * nbuf]
        dsem = rest[3 * nbuf:4 * nbuf]
        cid = lax.axis_index("c")
        sid = lax.axis_index("s")
        wid = cid * _NS + sid
        e_base = wid * e_per_w

        rs = pl.ds(sid * stride, wsize)

        @pl.when(cid == 0)
        def _():
            pltpu.sync_copy(h_hbm.at[rs], acc.at[rs])

        @pl.when(cid > 0)
        def _():
            pltpu.sync_copy(zero_hbm.at[rs], acc.at[rs])

        pltpu.sync_copy(src_hbm.at[pl.ds(e_base, e_per_w)], src_all)
        plsc.subcore_barrier()

        def fire(i, b):
            pltpu.async_copy(dst_hbm.at[pl.ds(e_base + i * K, K)], dstv[b], dsem[b])
            pltpu.async_copy(h_hbm.at[src_all.at[pl.ds(i * K, K)]], rows[b], gsem[b])

        def wait_and_scatter(i, b):
            pltpu.make_async_copy(h_hbm.at[src_all.at[pl.ds(i * K, K)]],
                                  rows[b], gsem[b]).wait()
            pltpu.make_async_copy(dst_hbm.at[pl.ds(e_base, K)], dstv[b],
                                  dsem[b]).wait()
            pltpu.sync_copy(rows[b], acc.at[dstv[b]], add=True)

        for b in range(nbuf):
            fire(b, b)

        main = niter - (niter % nbuf)

        @pl.loop(0, main, step=nbuf)
        def _(i):
            for b in range(nbuf):
                ii = i + b
                wait_and_scatter(ii, b)

                @pl.when(ii + nbuf < niter)
                def _():
                    fire(ii + nbuf, b)

        for r in range(niter % nbuf):
            wait_and_scatter(main + r, r)

        plsc.subcore_barrier()
        pltpu.sync_copy(acc.at[rs], out_hbm.at[pl.ds(cid * N + sid * stride, wsize)])

    return seg


@functools.lru_cache(maxsize=None)
def _make_mlp(N, D):
    def body(agg_ref, w1_ref, b1_ref, w2_ref, b2_ref, g_ref, be_ref,
             out_ref, mean_ref):
        z = agg_ref[:N, :] + agg_ref[N:, :]
        y = lax.dot_general(z, w1_ref[...], (((1,), (1,)), ((), ())),
                            precision=lax.Precision.HIGHEST,
                            preferred_element_type=jnp.float32) + b1_ref[...]
        mu = jnp.mean(y, axis=0, keepdims=True)
        var = jnp.mean((y - mu) * (y - mu), axis=0, keepdims=True)
        r = (y - mu) * lax.rsqrt(var + 1e-5) * g_ref[...] + be_ref[...]
        r = jnp.maximum(r, 0.0)
        o = lax.dot_general(r, w2_ref[...], (((1,), (1,)), ((), ())),
                            precision=lax.Precision.HIGHEST,
                            preferred_element_type=jnp.float32) + b2_ref[...]
        out_ref[...] = o
        mean_ref[...] = jnp.mean(o, axis=0, keepdims=True)

    return pl.pallas_call(
        body,
        out_shape=(jax.ShapeDtypeStruct((N, D), jnp.float32),
                   jax.ShapeDtypeStruct((1, D), jnp.float32)),
    )


def kernel(x, edge_index, W1, b1, W2, b2, gamma, beta):
    N, D = x.shape
    E = edge_index.shape[1]
    L = W1.shape[0]
    src = edge_index[0]
    dst = edge_index[1]
    zeros = jnp.zeros((N, D), jnp.float32)
    seg = _make_seg_sum(N, E, D)
    mlp = _make_mlp(N, D)

    h = x
    means = []
    for l in range(L):
        agg2 = seg(src, dst, h, zeros)
        h, m = mlp(agg2, W1[l], b1[l].reshape(1, D), W2[l], b2[l].reshape(1, D),
                   gamma[l].reshape(1, D), beta[l].reshape(1, D))
        means.append(m.reshape(D))
    return (means[-1], means[-2])


# speedup vs baseline: 11.6950x; 1.2038x over previous
import functools

import jax
import jax.numpy as jnp
from jax import lax
from jax.experimental import pallas as pl
from jax.experimental.pallas import tpu as pltpu
from jax.experimental.pallas import tpu_sc as plsc

_NC = 2
_NS = 16


@functools.lru_cache(maxsize=None)
def _make_seg_sum(N, E, D):
    NW = _NC * _NS
    e_per_w = E // NW
    K = 40
    niter = e_per_w // K
    assert e_per_w % K == 0
    stride = ((N // _NS) // 8) * 8
    wsize = N - (_NS - 1) * stride
    assert wsize % 8 == 0 and wsize >= stride and E % NW == 0

    mesh = plsc.VectorSubcoreMesh(core_axis_name="c", subcore_axis_name="s")

    nbuf = 5
    look = nbuf - 2
    assert niter % nbuf == 0 and niter >= nbuf
    scratch_types = [pltpu.VMEM((e_per_w,), jnp.int32),
                     pltpu.VMEM_SHARED((N, D), jnp.float32)]
    scratch_types += [pltpu.VMEM((K,), jnp.int32) for _ in range(nbuf)]
    scratch_types += [pltpu.VMEM((K, D), jnp.float32) for _ in range(nbuf)]
    scratch_types += [pltpu.SemaphoreType.DMA for _ in range(3 * nbuf)]

    @functools.partial(
        pl.kernel,
        out_type=jax.ShapeDtypeStruct((_NC * N, D), jnp.float32),
        mesh=mesh,
        scratch_types=scratch_types,
    )
    def seg(src_hbm, dst_hbm, h_hbm, zero_hbm, out_hbm, src_all, acc, *rest):
        dstv = rest[:nbuf]
        rows = rest[nbuf:2 * nbuf]
        gsem = rest[2 * nbuf:3 * nbuf]
        dsem = rest[3 * nbuf:4 * nbuf]
        ssem = rest[4 * nbuf:5 * nbuf]
        cid = lax.axis_index("c")
        sid = lax.axis_index("s")
        wid = cid * _NS + sid
        e_base = wid * e_per_w

        rs = pl.ds(sid * stride, wsize)

        @pl.when(cid == 0)
        def _():
            pltpu.sync_copy(h_hbm.at[rs], acc.at[rs])

        @pl.when(cid > 0)
        def _():
            pltpu.sync_copy(zero_hbm.at[rs], acc.at[rs])

        pltpu.sync_copy(src_hbm.at[pl.ds(e_base, e_per_w)], src_all)
        plsc.subcore_barrier()

        def fire(i, b):
            pltpu.async_copy(dst_hbm.at[pl.ds(e_base + i * K, K)], dstv[b], dsem[b])
            pltpu.async_copy(h_hbm.at[src_all.at[pl.ds(i * K, K)]], rows[b], gsem[b])

        def wait_scatter(b):
            pltpu.make_async_copy(rows[b], acc.at[dstv[b]], ssem[b]).wait()

        for c in range(look):
            fire(c, c)

        @pl.loop(0, niter, step=nbuf)
        def _(i):
            for b in range(nbuf):
                ii = i + b
                nb = (b + look) % nbuf

                @pl.when(ii >= 2)
                def _():
                    wait_scatter(nb)

                @pl.when(ii + look < niter)
                def _():
                    fire(ii + look, nb)

                pltpu.make_async_copy(h_hbm.at[src_all.at[pl.ds(ii * K, K)]],
                                      rows[b], gsem[b]).wait()
                pltpu.make_async_copy(dst_hbm.at[pl.ds(e_base, K)], dstv[b],
                                      dsem[b]).wait()
                pltpu.async_copy(rows[b], acc.at[dstv[b]], ssem[b])

        for c in range(niter - 2, niter):
            wait_scatter(c % nbuf)

        plsc.subcore_barrier()
        pltpu.sync_copy(acc.at[rs], out_hbm.at[pl.ds(cid * N + sid * stride, wsize)])

    return seg


@functools.lru_cache(maxsize=None)
def _make_mlp(N, D):
    def body(agg_ref, w1_ref, b1_ref, w2_ref, b2_ref, g_ref, be_ref,
             out_ref, mean_ref):
        z = agg_ref[:N, :] + agg_ref[N:, :]
        y = lax.dot_general(z, w1_ref[...], (((1,), (1,)), ((), ())),
                            precision=lax.Precision.HIGHEST,
                            preferred_element_type=jnp.float32) + b1_ref[...]
        mu = jnp.mean(y, axis=0, keepdims=True)
        var = jnp.mean((y - mu) * (y - mu), axis=0, keepdims=True)
        r = (y - mu) * lax.rsqrt(var + 1e-5) * g_ref[...] + be_ref[...]
        r = jnp.maximum(r, 0.0)
        o = lax.dot_general(r, w2_ref[...], (((1,), (1,)), ((), ())),
                            precision=lax.Precision.HIGHEST,
                            preferred_element_type=jnp.float32) + b2_ref[...]
        out_ref[...] = o
        mean_ref[...] = jnp.mean(o, axis=0, keepdims=True)

    return pl.pallas_call(
        body,
        out_shape=(jax.ShapeDtypeStruct((N, D), jnp.float32),
                   jax.ShapeDtypeStruct((1, D), jnp.float32)),
    )


def kernel(x, edge_index, W1, b1, W2, b2, gamma, beta):
    N, D = x.shape
    E = edge_index.shape[1]
    L = W1.shape[0]
    src = edge_index[0]
    dst = edge_index[1]
    zeros = jnp.zeros((N, D), jnp.float32)
    seg = _make_seg_sum(N, E, D)
    mlp = _make_mlp(N, D)

    h = x
    means = []
    for l in range(L):
        agg2 = seg(src, dst, h, zeros)
        h, m = mlp(agg2, W1[l], b1[l].reshape(1, D), W2[l], b2[l].reshape(1, D),
                   gamma[l].reshape(1, D), beta[l].reshape(1, D))
        means.append(m.reshape(D))
    return (means[-1], means[-2])


# speedup vs baseline: 13.3276x; 1.1396x over previous
import functools

import jax
import jax.numpy as jnp
from jax import lax
from jax.experimental import pallas as pl
from jax.experimental.pallas import tpu as pltpu
from jax.experimental.pallas import tpu_sc as plsc

_NC = 2
_NS = 16


@functools.lru_cache(maxsize=None)
def _make_seg_sum(N, E, D):
    NW = _NC * _NS
    e_per_w = E // NW
    K = 40
    niter = e_per_w // K
    assert e_per_w % K == 0
    stride = ((N // _NS) // 8) * 8
    wsize = N - (_NS - 1) * stride
    assert wsize % 8 == 0 and wsize >= stride and E % NW == 0

    mesh = plsc.VectorSubcoreMesh(core_axis_name="c", subcore_axis_name="s")

    nbuf = 5
    look = nbuf - 2
    assert niter % nbuf == 0 and niter >= nbuf
    scratch_types = [pltpu.VMEM((e_per_w,), jnp.int32),
                     pltpu.VMEM_SHARED((N, D), jnp.float32)]
    scratch_types += [pltpu.VMEM((K,), jnp.int32) for _ in range(nbuf)]
    scratch_types += [pltpu.VMEM((K, D), jnp.float32) for _ in range(nbuf)]
    scratch_types += [pltpu.SemaphoreType.DMA for _ in range(3 * nbuf)]

    @functools.partial(
        pl.kernel,
        out_type=jax.ShapeDtypeStruct((_NC * N, D), jnp.float32),
        mesh=mesh,
        scratch_types=scratch_types,
    )
    def seg(src_hbm, dst_hbm, h_hbm, zero_hbm, out_hbm, src_all, acc, *rest):
        dstv = rest[:nbuf]
        rows = rest[nbuf:2 * nbuf]
        gsem = rest[2 * nbuf:3 * nbuf]
        dsem = rest[3 * nbuf:4 * nbuf]
        ssem = rest[4 * nbuf:5 * nbuf]
        cid = lax.axis_index("c")
        sid = lax.axis_index("s")
        wid = cid * _NS + sid
        e_base = wid * e_per_w

        rs = pl.ds(sid * stride, wsize)

        @pl.when(cid == 0)
        def _():
            pltpu.sync_copy(h_hbm.at[rs], acc.at[rs])

        @pl.when(cid > 0)
        def _():
            pltpu.sync_copy(zero_hbm.at[rs], acc.at[rs])

        pltpu.sync_copy(src_hbm.at[pl.ds(e_base, e_per_w)], src_all)
        plsc.subcore_barrier()

        def fire(i, b):
            pltpu.async_copy(dst_hbm.at[pl.ds(e_base + i * K, K)], dstv[b], dsem[b])
            pltpu.async_copy(h_hbm.at[src_all.at[pl.ds(i * K, K)]], rows[b], gsem[b])

        def wait_scatter(b):
            pltpu.make_async_copy(rows[b], acc.at[dstv[b]], ssem[b]).wait()

        for c in range(look):
            fire(c, c)

        @pl.loop(0, niter, step=nbuf)
        def _(i):
            for b in range(nbuf):
                ii = i + b
                nb = (b + look) % nbuf

                @pl.when(ii >= 2)
                def _():
                    wait_scatter(nb)

                @pl.when(ii + look < niter)
                def _():
                    fire(ii + look, nb)

                pltpu.make_async_copy(h_hbm.at[src_all.at[pl.ds(ii * K, K)]],
                                      rows[b], gsem[b]).wait()
                pltpu.make_async_copy(dst_hbm.at[pl.ds(e_base, K)], dstv[b],
                                      dsem[b]).wait()
                pltpu.async_copy(rows[b], acc.at[dstv[b]], ssem[b])

        for c in range(niter - 2, niter):
            wait_scatter(c % nbuf)

        plsc.subcore_barrier()
        pltpu.sync_copy(acc.at[rs], out_hbm.at[pl.ds(cid * N + sid * stride, wsize)])

    return seg


@functools.lru_cache(maxsize=None)
def _make_mlp(N, D):
    def body(agg_ref, w1_ref, b1_ref, w2_ref, b2_ref, g_ref, be_ref,
             out_ref, mean_ref):
        z = agg_ref[:N, :] + agg_ref[N:, :]
        y = lax.dot_general(z, w1_ref[...], (((1,), (1,)), ((), ())),
                            precision=lax.Precision.DEFAULT,
                            preferred_element_type=jnp.float32) + b1_ref[...]
        mu = jnp.mean(y, axis=0, keepdims=True)
        var = jnp.mean((y - mu) * (y - mu), axis=0, keepdims=True)
        r = (y - mu) * lax.rsqrt(var + 1e-5) * g_ref[...] + be_ref[...]
        r = jnp.maximum(r, 0.0)
        o = lax.dot_general(r, w2_ref[...], (((1,), (1,)), ((), ())),
                            precision=lax.Precision.DEFAULT,
                            preferred_element_type=jnp.float32) + b2_ref[...]
        out_ref[...] = o
        mean_ref[...] = jnp.mean(o, axis=0, keepdims=True)

    return pl.pallas_call(
        body,
        out_shape=(jax.ShapeDtypeStruct((N, D), jnp.float32),
                   jax.ShapeDtypeStruct((1, D), jnp.float32)),
    )


def kernel(x, edge_index, W1, b1, W2, b2, gamma, beta):
    N, D = x.shape
    E = edge_index.shape[1]
    L = W1.shape[0]
    src = edge_index[0]
    dst = edge_index[1]
    zeros = jnp.zeros((N, D), jnp.float32)
    seg = _make_seg_sum(N, E, D)
    mlp = _make_mlp(N, D)

    h = x
    means = []
    for l in range(L):
        agg2 = seg(src, dst, h, zeros)
        h, m = mlp(agg2, W1[l], b1[l].reshape(1, D), W2[l], b2[l].reshape(1, D),
                   gamma[l].reshape(1, D), beta[l].reshape(1, D))
        means.append(m.reshape(D))
    return (means[-1], means[-2])


# speedup vs baseline: 13.5844x; 1.0193x over previous
import functools

import jax
import jax.numpy as jnp
from jax import lax
from jax.experimental import pallas as pl
from jax.experimental.pallas import tpu as pltpu
from jax.experimental.pallas import tpu_sc as plsc

_NC = 2
_NS = 16


@functools.lru_cache(maxsize=None)
def _make_seg_sum(N, E, D):
    NW = _NC * _NS
    e_per_w = E // NW
    K = 40
    niter = e_per_w // K
    assert e_per_w % K == 0
    stride = ((N // _NS) // 8) * 8
    wsize = N - (_NS - 1) * stride
    assert wsize % 8 == 0 and wsize >= stride and E % NW == 0

    mesh = plsc.VectorSubcoreMesh(core_axis_name="c", subcore_axis_name="s")

    nbuf = 5
    look = nbuf - 2
    assert niter % nbuf == 0 and niter >= nbuf
    scratch_types = [pltpu.VMEM((e_per_w,), jnp.int32),
                     pltpu.VMEM_SHARED((N, D), jnp.float32)]
    scratch_types += [pltpu.VMEM((K,), jnp.int32) for _ in range(nbuf)]
    scratch_types += [pltpu.VMEM((K, D), jnp.float32) for _ in range(nbuf)]
    scratch_types += [pltpu.SemaphoreType.DMA for _ in range(3 * nbuf + 1)]

    @functools.partial(
        pl.kernel,
        out_type=jax.ShapeDtypeStruct((_NC * N, D), jnp.float32),
        mesh=mesh,
        scratch_types=scratch_types,
    )
    def seg(src_hbm, dst_hbm, h_hbm, zero_hbm, out_hbm, src_all, acc, *rest):
        dstv = rest[:nbuf]
        rows = rest[nbuf:2 * nbuf]
        gsem = rest[2 * nbuf:3 * nbuf]
        dsem = rest[3 * nbuf:4 * nbuf]
        ssem = rest[4 * nbuf:5 * nbuf]
        isem = rest[5 * nbuf]
        cid = lax.axis_index("c")
        sid = lax.axis_index("s")
        wid = cid * _NS + sid
        e_base = wid * e_per_w

        rs = pl.ds(sid * stride, wsize)

        @pl.when(cid == 0)
        def _():
            pltpu.async_copy(h_hbm.at[rs], acc.at[rs], isem)

        @pl.when(cid > 0)
        def _():
            pltpu.async_copy(zero_hbm.at[rs], acc.at[rs], isem)

        pltpu.sync_copy(src_hbm.at[pl.ds(e_base, e_per_w)], src_all)

        def fire(i, b):
            pltpu.async_copy(dst_hbm.at[pl.ds(e_base + i * K, K)], dstv[b], dsem[b])
            pltpu.async_copy(h_hbm.at[src_all.at[pl.ds(i * K, K)]], rows[b], gsem[b])

        def wait_scatter(b):
            pltpu.make_async_copy(rows[b], acc.at[dstv[b]], ssem[b]).wait()

        for c in range(look):
            fire(c, c)

        pltpu.make_async_copy(zero_hbm.at[rs], acc.at[rs], isem).wait()
        plsc.subcore_barrier()

        @pl.loop(0, niter, step=nbuf)
        def _(i):
            for b in range(nbuf):
                ii = i + b
                nb = (b + look) % nbuf

                @pl.when(ii >= 2)
                def _():
                    wait_scatter(nb)

                @pl.when(ii + look < niter)
                def _():
                    fire(ii + look, nb)

                pltpu.make_async_copy(h_hbm.at[src_all.at[pl.ds(ii * K, K)]],
                                      rows[b], gsem[b]).wait()
                pltpu.make_async_copy(dst_hbm.at[pl.ds(e_base, K)], dstv[b],
                                      dsem[b]).wait()
                pltpu.async_copy(rows[b], acc.at[dstv[b]], ssem[b])

        for c in range(niter - 2, niter):
            wait_scatter(c % nbuf)

        plsc.subcore_barrier()
        pltpu.sync_copy(acc.at[rs], out_hbm.at[pl.ds(cid * N + sid * stride, wsize)])

    return seg


@functools.lru_cache(maxsize=None)
def _make_mlp(N, D):
    def body(agg_ref, w1_ref, b1_ref, w2_ref, b2_ref, g_ref, be_ref,
             out_ref, mean_ref):
        z = agg_ref[:N, :] + agg_ref[N:, :]
        y = lax.dot_general(z, w1_ref[...], (((1,), (1,)), ((), ())),
                            precision=lax.Precision.DEFAULT,
                            preferred_element_type=jnp.float32) + b1_ref[...]
        mu = jnp.mean(y, axis=0, keepdims=True)
        var = jnp.mean((y - mu) * (y - mu), axis=0, keepdims=True)
        r = (y - mu) * lax.rsqrt(var + 1e-5) * g_ref[...] + be_ref[...]
        r = jnp.maximum(r, 0.0)
        o = lax.dot_general(r, w2_ref[...], (((1,), (1,)), ((), ())),
                            precision=lax.Precision.DEFAULT,
                            preferred_element_type=jnp.float32) + b2_ref[...]
        out_ref[...] = o
        mean_ref[...] = jnp.mean(o, axis=0, keepdims=True)

    return pl.pallas_call(
        body,
        out_shape=(jax.ShapeDtypeStruct((N, D), jnp.float32),
                   jax.ShapeDtypeStruct((1, D), jnp.float32)),
    )


def kernel(x, edge_index, W1, b1, W2, b2, gamma, beta):
    N, D = x.shape
    E = edge_index.shape[1]
    L = W1.shape[0]
    src = edge_index[0]
    dst = edge_index[1]
    zeros = jnp.zeros((N, D), jnp.float32)
    seg = _make_seg_sum(N, E, D)
    mlp = _make_mlp(N, D)

    h = x
    means = []
    for l in range(L):
        agg2 = seg(src, dst, h, zeros)
        h, m = mlp(agg2, W1[l], b1[l].reshape(1, D), W2[l], b2[l].reshape(1, D),
                   gamma[l].reshape(1, D), beta[l].reshape(1, D))
        means.append(m.reshape(D))
    return (means[-1], means[-2])


# speedup vs baseline: 14.1412x; 1.0410x over previous
import functools

import jax
import jax.numpy as jnp
from jax import lax
from jax.experimental import pallas as pl
from jax.experimental.pallas import tpu as pltpu
from jax.experimental.pallas import tpu_sc as plsc

_NC = 2
_NS = 16


@functools.lru_cache(maxsize=None)
def _make_seg_sum(N, E, D):
    NW = _NC * _NS
    e_per_w = E // NW
    K = 40
    niter = e_per_w // K
    assert e_per_w % K == 0
    stride = ((N // _NS) // 8) * 8
    wsize = N - (_NS - 1) * stride
    assert wsize % 8 == 0 and wsize >= stride and E % NW == 0

    mesh = plsc.VectorSubcoreMesh(core_axis_name="c", subcore_axis_name="s")

    nbuf = 5
    look = 4
    assert niter % nbuf == 0 and niter >= nbuf
    scratch_types = [pltpu.VMEM((e_per_w,), jnp.int32),
                     pltpu.VMEM_SHARED((N, D), jnp.float32)]
    scratch_types += [pltpu.VMEM((K,), jnp.int32) for _ in range(nbuf)]
    scratch_types += [pltpu.VMEM((K, D), jnp.float32) for _ in range(nbuf)]
    scratch_types += [pltpu.SemaphoreType.DMA for _ in range(3 * nbuf + 1)]

    @functools.partial(
        pl.kernel,
        out_type=jax.ShapeDtypeStruct((_NC * N, D), jnp.float32),
        mesh=mesh,
        scratch_types=scratch_types,
    )
    def seg(src_hbm, dst_hbm, h_hbm, zero_hbm, out_hbm, src_all, acc, *rest):
        dstv = rest[:nbuf]
        rows = rest[nbuf:2 * nbuf]
        gsem = rest[2 * nbuf:3 * nbuf]
        dsem = rest[3 * nbuf:4 * nbuf]
        ssem = rest[4 * nbuf:5 * nbuf]
        isem = rest[5 * nbuf]
        cid = lax.axis_index("c")
        sid = lax.axis_index("s")
        wid = cid * _NS + sid
        e_base = wid * e_per_w

        rs = pl.ds(sid * stride, wsize)

        @pl.when(cid == 0)
        def _():
            pltpu.async_copy(h_hbm.at[rs], acc.at[rs], isem)

        @pl.when(cid > 0)
        def _():
            pltpu.async_copy(zero_hbm.at[rs], acc.at[rs], isem)

        pltpu.sync_copy(src_hbm.at[pl.ds(e_base, e_per_w)], src_all)

        def fire(i, b):
            pltpu.async_copy(dst_hbm.at[pl.ds(e_base + i * K, K)], dstv[b], dsem[b])
            pltpu.async_copy(h_hbm.at[src_all.at[pl.ds(i * K, K)]], rows[b], gsem[b])

        def wait_scatter(b):
            pltpu.make_async_copy(rows[b], acc.at[dstv[b]], ssem[b]).wait()

        for c in range(look):
            fire(c, c)

        pltpu.make_async_copy(zero_hbm.at[rs], acc.at[rs], isem).wait()
        plsc.subcore_barrier()

        @pl.loop(0, niter, step=nbuf)
        def _(i):
            for b in range(nbuf):
                ii = i + b
                nb = (b + look) % nbuf

                @pl.when(ii >= nbuf - look)
                def _():
                    wait_scatter(nb)

                @pl.when(ii + look < niter)
                def _():
                    fire(ii + look, nb)

                pltpu.make_async_copy(h_hbm.at[src_all.at[pl.ds(ii * K, K)]],
                                      rows[b], gsem[b]).wait()
                pltpu.make_async_copy(dst_hbm.at[pl.ds(e_base, K)], dstv[b],
                                      dsem[b]).wait()
                pltpu.async_copy(rows[b], acc.at[dstv[b]], ssem[b])

        for c in range(niter - (nbuf - look), niter):
            wait_scatter(c % nbuf)

        plsc.subcore_barrier()
        pltpu.sync_copy(acc.at[rs], out_hbm.at[pl.ds(cid * N + sid * stride, wsize)])

    return seg


@functools.lru_cache(maxsize=None)
def _make_mlp(N, D):
    def body(agg_ref, w1_ref, b1_ref, w2_ref, b2_ref, g_ref, be_ref,
             out_ref, mean_ref):
        z = agg_ref[:N, :] + agg_ref[N:, :]
        y = lax.dot_general(z, w1_ref[...], (((1,), (1,)), ((), ())),
                            precision=lax.Precision.DEFAULT,
                            preferred_element_type=jnp.float32) + b1_ref[...]
        mu = jnp.mean(y, axis=0, keepdims=True)
        var = jnp.mean((y - mu) * (y - mu), axis=0, keepdims=True)
        r = (y - mu) * lax.rsqrt(var + 1e-5) * g_ref[...] + be_ref[...]
        r = jnp.maximum(r, 0.0)
        o = lax.dot_general(r, w2_ref[...], (((1,), (1,)), ((), ())),
                            precision=lax.Precision.DEFAULT,
                            preferred_element_type=jnp.float32) + b2_ref[...]
        out_ref[...] = o
        mean_ref[...] = jnp.mean(o, axis=0, keepdims=True)

    return pl.pallas_call(
        body,
        out_shape=(jax.ShapeDtypeStruct((N, D), jnp.float32),
                   jax.ShapeDtypeStruct((1, D), jnp.float32)),
    )


def kernel(x, edge_index, W1, b1, W2, b2, gamma, beta):
    N, D = x.shape
    E = edge_index.shape[1]
    L = W1.shape[0]
    src = edge_index[0]
    dst = edge_index[1]
    zeros = jnp.zeros((N, D), jnp.float32)
    seg = _make_seg_sum(N, E, D)
    mlp = _make_mlp(N, D)

    h = x
    means = []
    for l in range(L):
        agg2 = seg(src, dst, h, zeros)
        h, m = mlp(agg2, W1[l], b1[l].reshape(1, D), W2[l], b2[l].reshape(1, D),
                   gamma[l].reshape(1, D), beta[l].reshape(1, D))
        means.append(m.reshape(D))
    return (means[-1], means[-2])


# speedup vs baseline: 14.7922x; 1.0460x over previous
import functools

import jax
import jax.numpy as jnp
from jax import lax
from jax.experimental import pallas as pl
from jax.experimental.pallas import tpu as pltpu
from jax.experimental.pallas import tpu_sc as plsc

_NC = 2
_NS = 16


@functools.lru_cache(maxsize=None)
def _make_seg_sum(N, E, D):
    NW = _NC * _NS
    e_per_w = E // NW
    K = 40
    niter = e_per_w // K
    assert e_per_w % K == 0
    stride = ((N // _NS) // 8) * 8
    wsize = N - (_NS - 1) * stride
    assert wsize % 8 == 0 and wsize >= stride and E % NW == 0

    mesh = plsc.VectorSubcoreMesh(core_axis_name="c", subcore_axis_name="s")

    nbuf = 6
    look = 5
    assert niter >= 2 * nbuf
    scratch_types = [pltpu.VMEM((e_per_w,), jnp.int32),
                     pltpu.VMEM_SHARED((N, D), jnp.float32)]
    scratch_types += [pltpu.VMEM((K,), jnp.int32) for _ in range(nbuf)]
    scratch_types += [pltpu.VMEM((K, D), jnp.float32) for _ in range(nbuf)]
    scratch_types += [pltpu.SemaphoreType.DMA for _ in range(3 * nbuf + 1)]

    @functools.partial(
        pl.kernel,
        out_type=jax.ShapeDtypeStruct((_NC * N, D), jnp.float32),
        mesh=mesh,
        scratch_types=scratch_types,
    )
    def seg(src_hbm, dst_hbm, h_hbm, zero_hbm, out_hbm, src_all, acc, *rest):
        dstv = rest[:nbuf]
        rows = rest[nbuf:2 * nbuf]
        gsem = rest[2 * nbuf:3 * nbuf]
        dsem = rest[3 * nbuf:4 * nbuf]
        ssem = rest[4 * nbuf:5 * nbuf]
        isem = rest[5 * nbuf]
        cid = lax.axis_index("c")
        sid = lax.axis_index("s")
        wid = cid * _NS + sid
        e_base = wid * e_per_w

        rs = pl.ds(sid * stride, wsize)

        @pl.when(cid == 0)
        def _():
            pltpu.async_copy(h_hbm.at[rs], acc.at[rs], isem)

        @pl.when(cid > 0)
        def _():
            pltpu.async_copy(zero_hbm.at[rs], acc.at[rs], isem)

        pltpu.sync_copy(src_hbm.at[pl.ds(e_base, e_per_w)], src_all)

        def fire(i, b):
            pltpu.async_copy(dst_hbm.at[pl.ds(e_base + i * K, K)], dstv[b], dsem[b])
            pltpu.async_copy(h_hbm.at[src_all.at[pl.ds(i * K, K)]], rows[b], gsem[b])

        def wait_scatter(b):
            pltpu.make_async_copy(rows[b], acc.at[dstv[b]], ssem[b]).wait()

        for c in range(look):
            fire(c, c)

        pltpu.make_async_copy(zero_hbm.at[rs], acc.at[rs], isem).wait()
        plsc.subcore_barrier()

        def step(ii, b):
            nb = (b + look) % nbuf

            @pl.when(ii >= nbuf - look)
            def _():
                wait_scatter(nb)

            @pl.when(ii + look < niter)
            def _():
                fire(ii + look, nb)

            pltpu.make_async_copy(h_hbm.at[src_all.at[pl.ds(ii * K, K)]],
                                  rows[b], gsem[b]).wait()
            pltpu.make_async_copy(dst_hbm.at[pl.ds(e_base, K)], dstv[b],
                                  dsem[b]).wait()
            pltpu.async_copy(rows[b], acc.at[dstv[b]], ssem[b])

        main = niter - (niter % nbuf)

        @pl.loop(0, main, step=nbuf)
        def _(i):
            for b in range(nbuf):
                step(i + b, b)

        for c in range(main, niter):
            step(c, c % nbuf)

        for c in range(niter - (nbuf - look), niter):
            wait_scatter(c % nbuf)

        plsc.subcore_barrier()
        pltpu.sync_copy(acc.at[rs], out_hbm.at[pl.ds(cid * N + sid * stride, wsize)])

    return seg


@functools.lru_cache(maxsize=None)
def _make_mlp(N, D):
    def body(agg_ref, w1_ref, b1_ref, w2_ref, b2_ref, g_ref, be_ref,
             out_ref, mean_ref):
        z = agg_ref[:N, :] + agg_ref[N:, :]
        y = lax.dot_general(z, w1_ref[...], (((1,), (1,)), ((), ())),
                            precision=lax.Precision.DEFAULT,
                            preferred_element_type=jnp.float32) + b1_ref[...]
        mu = jnp.mean(y, axis=0, keepdims=True)
        var = jnp.mean((y - mu) * (y - mu), axis=0, keepdims=True)
        r = (y - mu) * lax.rsqrt(var + 1e-5) * g_ref[...] + be_ref[...]
        r = jnp.maximum(r, 0.0)
        o = lax.dot_general(r, w2_ref[...], (((1,), (1,)), ((), ())),
                            precision=lax.Precision.DEFAULT,
                            preferred_element_type=jnp.float32) + b2_ref[...]
        out_ref[...] = o
        mean_ref[...] = jnp.mean(o, axis=0, keepdims=True)

    return pl.pallas_call(
        body,
        out_shape=(jax.ShapeDtypeStruct((N, D), jnp.float32),
                   jax.ShapeDtypeStruct((1, D), jnp.float32)),
    )


def kernel(x, edge_index, W1, b1, W2, b2, gamma, beta):
    N, D = x.shape
    E = edge_index.shape[1]
    L = W1.shape[0]
    src = edge_index[0]
    dst = edge_index[1]
    zeros = jnp.zeros((N, D), jnp.float32)
    seg = _make_seg_sum(N, E, D)
    mlp = _make_mlp(N, D)

    h = x
    means = []
    for l in range(L):
        agg2 = seg(src, dst, h, zeros)
        h, m = mlp(agg2, W1[l], b1[l].reshape(1, D), W2[l], b2[l].reshape(1, D),
                   gamma[l].reshape(1, D), beta[l].reshape(1, D))
        means.append(m.reshape(D))
    return (means[-1], means[-2])


# speedup vs baseline: 15.0129x; 1.0149x over previous
import functools

import jax
import jax.numpy as jnp
from jax import lax
from jax.experimental import pallas as pl
from jax.experimental.pallas import tpu as pltpu
from jax.experimental.pallas import tpu_sc as plsc

_NC = 2
_NS = 16


@functools.lru_cache(maxsize=None)
def _make_seg_sum(N, E, D):
    NW = _NC * _NS
    e_per_w = E // NW
    K = 40
    niter = e_per_w // K
    assert e_per_w % K == 0
    stride = ((N // _NS) // 8) * 8
    wsize = N - (_NS - 1) * stride
    assert wsize % 8 == 0 and wsize >= stride and E % NW == 0

    mesh = plsc.VectorSubcoreMesh(core_axis_name="c", subcore_axis_name="s")

    nbuf = 7
    look = 6
    assert niter >= 2 * nbuf
    scratch_types = [pltpu.VMEM((e_per_w,), jnp.int32),
                     pltpu.VMEM_SHARED((N, D), jnp.float32)]
    scratch_types += [pltpu.VMEM((K,), jnp.int32) for _ in range(nbuf)]
    scratch_types += [pltpu.VMEM((K, D), jnp.float32) for _ in range(nbuf)]
    scratch_types += [pltpu.SemaphoreType.DMA for _ in range(3 * nbuf + 1)]

    @functools.partial(
        pl.kernel,
        out_type=jax.ShapeDtypeStruct((_NC * N, D), jnp.float32),
        mesh=mesh,
        scratch_types=scratch_types,
    )
    def seg(src_hbm, dst_hbm, h_hbm, zero_hbm, out_hbm, src_all, acc, *rest):
        dstv = rest[:nbuf]
        rows = rest[nbuf:2 * nbuf]
        gsem = rest[2 * nbuf:3 * nbuf]
        dsem = rest[3 * nbuf:4 * nbuf]
        ssem = rest[4 * nbuf:5 * nbuf]
        isem = rest[5 * nbuf]
        cid = lax.axis_index("c")
        sid = lax.axis_index("s")
        wid = cid * _NS + sid
        e_base = wid * e_per_w

        rs = pl.ds(sid * stride, wsize)

        @pl.when(cid == 0)
        def _():
            pltpu.async_copy(h_hbm.at[rs], acc.at[rs], isem)

        @pl.when(cid > 0)
        def _():
            pltpu.async_copy(zero_hbm.at[rs], acc.at[rs], isem)

        pltpu.sync_copy(src_hbm.at[pl.ds(e_base, e_per_w)], src_all)

        def fire(i, b):
            pltpu.async_copy(dst_hbm.at[pl.ds(e_base + i * K, K)], dstv[b], dsem[b])
            pltpu.async_copy(h_hbm.at[src_all.at[pl.ds(i * K, K)]], rows[b], gsem[b])

        def wait_scatter(b):
            pltpu.make_async_copy(rows[b], acc.at[dstv[b]], ssem[b]).wait()

        for c in range(look):
            fire(c, c)

        pltpu.make_async_copy(zero_hbm.at[rs], acc.at[rs], isem).wait()
        plsc.subcore_barrier()

        def step(ii, b):
            nb = (b + look) % nbuf

            @pl.when(ii >= nbuf - look)
            def _():
                wait_scatter(nb)

            @pl.when(ii + look < niter)
            def _():
                fire(ii + look, nb)

            pltpu.make_async_copy(h_hbm.at[src_all.at[pl.ds(ii * K, K)]],
                                  rows[b], gsem[b]).wait()
            pltpu.make_async_copy(dst_hbm.at[pl.ds(e_base, K)], dstv[b],
                                  dsem[b]).wait()
            pltpu.async_copy(rows[b], acc.at[dstv[b]], ssem[b])

        main = niter - (niter % nbuf)

        @pl.loop(0, main, step=nbuf)
        def _(i):
            for b in range(nbuf):
                step(i + b, b)

        for c in range(main, niter):
            step(c, c % nbuf)

        for c in range(niter - (nbuf - look), niter):
            wait_scatter(c % nbuf)

        plsc.subcore_barrier()
        pltpu.sync_copy(acc.at[rs], out_hbm.at[pl.ds(cid * N + sid * stride, wsize)])

    return seg


@functools.lru_cache(maxsize=None)
def _make_mlp(N, D):
    def body(agg_ref, w1_ref, b1_ref, w2_ref, b2_ref, g_ref, be_ref,
             out_ref, mean_ref):
        z = agg_ref[:N, :] + agg_ref[N:, :]
        y = lax.dot_general(z, w1_ref[...], (((1,), (1,)), ((), ())),
                            precision=lax.Precision.DEFAULT,
                            preferred_element_type=jnp.float32) + b1_ref[...]
        mu = jnp.mean(y, axis=0, keepdims=True)
        var = jnp.mean((y - mu) * (y - mu), axis=0, keepdims=True)
        r = (y - mu) * lax.rsqrt(var + 1e-5) * g_ref[...] + be_ref[...]
        r = jnp.maximum(r, 0.0)
        o = lax.dot_general(r, w2_ref[...], (((1,), (1,)), ((), ())),
                            precision=lax.Precision.DEFAULT,
                            preferred_element_type=jnp.float32) + b2_ref[...]
        out_ref[...] = o
        mean_ref[...] = jnp.mean(o, axis=0, keepdims=True)

    return pl.pallas_call(
        body,
        out_shape=(jax.ShapeDtypeStruct((N, D), jnp.float32),
                   jax.ShapeDtypeStruct((1, D), jnp.float32)),
    )


def kernel(x, edge_index, W1, b1, W2, b2, gamma, beta):
    N, D = x.shape
    E = edge_index.shape[1]
    L = W1.shape[0]
    src = edge_index[0]
    dst = edge_index[1]
    zeros = jnp.zeros((N, D), jnp.float32)
    seg = _make_seg_sum(N, E, D)
    mlp = _make_mlp(N, D)

    h = x
    means = []
    for l in range(L):
        agg2 = seg(src, dst, h, zeros)
        h, m = mlp(agg2, W1[l], b1[l].reshape(1, D), W2[l], b2[l].reshape(1, D),
                   gamma[l].reshape(1, D), beta[l].reshape(1, D))
        means.append(m.reshape(D))
    return (means[-1], means[-2])
